# baseline (speedup 1.0000x reference)
import functools

import jax
import jax.numpy as jnp
from jax import lax
from jax.experimental import pallas as pl
from jax.experimental.pallas import tpu as pltpu
from jax.experimental.pallas import tpu_sc as plsc

VOCAB = 1000000
EMBED = 64
BATCH = 16384

NUM_CORES = 2
NUM_SUBCORES = 16
LANES = 16
NW = NUM_CORES * NUM_SUBCORES
BPW = BATCH // NW
CHUNK = 128
NCHUNK = BPW // CHUNK
GROUPS = BPW // LANES

_mesh = plsc.VectorSubcoreMesh(core_axis_name="c", subcore_axis_name="s")


@functools.partial(
    pl.kernel,
    mesh=_mesh,
    out_type=jax.ShapeDtypeStruct((BATCH,), jnp.float32),
    scratch_types=[
        pltpu.VMEM((NCHUNK, CHUNK), jnp.int32),
        pltpu.VMEM((NCHUNK, CHUNK), jnp.int32),
        pltpu.VMEM((BPW, EMBED), jnp.float32),
        pltpu.VMEM((BPW, EMBED), jnp.float32),
        pltpu.VMEM((BPW + LANES,), jnp.float32),
        pltpu.SemaphoreType.DMA,
    ],
    compiler_params=pltpu.CompilerParams(needs_layout_passes=False,
                                         use_tc_tiling_on_sc=False),
)
def _w2v_sc(center_hbm, context_hbm, in_emb_hbm, out_emb_hbm, out_hbm,
            cidx_v, xidx_v, arows_v, brows_v, out_v, sem):
    wid = lax.axis_index("s") * NUM_CORES + lax.axis_index("c")
    base = wid * BPW

    for k in range(NCHUNK):
        pltpu.sync_copy(center_hbm.at[pl.ds(base + k * CHUNK, CHUNK)],
                        cidx_v.at[k])
        pltpu.sync_copy(context_hbm.at[pl.ds(base + k * CHUNK, CHUNK)],
                        xidx_v.at[k])

    copies = []
    for k in range(NCHUNK):
        dst = arows_v.at[pl.ds(k * CHUNK, CHUNK)]
        copies.append(pltpu.async_copy(in_emb_hbm.at[cidx_v.at[k]], dst, sem))
        dst = brows_v.at[pl.ds(k * CHUNK, CHUNK)]
        copies.append(pltpu.async_copy(out_emb_hbm.at[xidx_v.at[k]], dst, sem))
    for c in copies:
        c.wait()

    last_lane = lax.iota(jnp.int32, LANES) == (LANES - 1)

    def row_body(r, _):
        s = jnp.zeros((LANES,), jnp.float32)
        for k in range(EMBED // LANES):
            a = arows_v[r, pl.ds(k * LANES, LANES)]
            b = brows_v[r, pl.ds(k * LANES, LANES)]
            s = s + a * b
        plsc.store_compressed(out_v.at[pl.ds(r, LANES)], plsc.cumsum(s),
                              mask=last_lane)
        return 0

    lax.fori_loop(0, BPW, row_body, 0, unroll=4)

    pltpu.sync_copy(out_v.at[pl.ds(0, BPW)], out_hbm.at[pl.ds(base, BPW)])


def kernel(center_idx, context_idx, in_embedding, out_embedding):
    return _w2v_sc(center_idx.astype(jnp.int32), context_idx.astype(jnp.int32),
                   in_embedding, out_embedding)


# speedup vs baseline: 1.5719x; 1.5719x over previous
import functools

import jax
import jax.numpy as jnp
from jax import lax
from jax.experimental import pallas as pl
from jax.experimental.pallas import tpu as pltpu
from jax.experimental.pallas import tpu_sc as plsc

VOCAB = 1000000
EMBED = 64
BATCH = 16384

NUM_CORES = 2
NUM_SUBCORES = 16
LANES = 16
NW = NUM_CORES * NUM_SUBCORES
BPW = BATCH // NW
CH = 128
NCHUNK = BPW // CH

_mesh = plsc.VectorSubcoreMesh(core_axis_name="c", subcore_axis_name="s")


@functools.partial(
    pl.kernel,
    mesh=_mesh,
    out_type=jax.ShapeDtypeStruct((BATCH,), jnp.float32),
    scratch_types=[
        pltpu.VMEM((BPW,), jnp.int32),
        pltpu.VMEM((BPW,), jnp.int32),
        pltpu.VMEM((2, CH // 8, 8, EMBED), jnp.float32),
        pltpu.VMEM((2, CH // 8, 8, EMBED), jnp.float32),
        pltpu.VMEM((BPW + LANES,), jnp.float32),
        pltpu.SemaphoreType.DMA,
        pltpu.SemaphoreType.DMA,
    ],
    compiler_params=pltpu.CompilerParams(needs_layout_passes=False),
)
def _w2v_sc(center_hbm, context_hbm, in_emb_hbm, out_emb_hbm, out_hbm,
            cidx_s, xidx_s, arows_v, brows_v, out_v, sem0, sem1):
    wid = lax.axis_index("s") * NUM_CORES + lax.axis_index("c")
    base = wid * BPW

    pltpu.sync_copy(center_hbm.at[pl.ds(base, BPW)], cidx_s)
    pltpu.sync_copy(context_hbm.at[pl.ds(base, BPW)], xidx_s)

    sems = [sem0, sem1]
    last_lane = lax.iota(jnp.int32, LANES) == (LANES - 1)

    def issue_chunk(c, buf):
        sem = sems[buf]

        def issue_group(g, _):
            r0 = c * CH + g * LANES
            av = cidx_s[pl.ds(r0, LANES)]
            bv = xidx_s[pl.ds(r0, LANES)]
            for k in range(LANES):
                i = g * LANES + k
                t, s = i // 8, i % 8
                pltpu.async_copy(in_emb_hbm.at[pl.ds(av[k], 1)],
                                 arows_v.at[buf, t, pl.ds(s, 1)], sem)
                pltpu.async_copy(out_emb_hbm.at[pl.ds(bv[k], 1)],
                                 brows_v.at[buf, t, pl.ds(s, 1)], sem)
            return 0

        lax.fori_loop(0, CH // LANES, issue_group, 0)

    def drain_chunk(buf):
        sem = sems[buf]

        def drain_row(i, _):
            t, s = i // 8, i % 8
            pltpu.make_async_copy(in_emb_hbm.at[pl.ds(0, 1)],
                                  arows_v.at[buf, t, pl.ds(s, 1)], sem).wait()
            pltpu.make_async_copy(out_emb_hbm.at[pl.ds(0, 1)],
                                  brows_v.at[buf, t, pl.ds(s, 1)], sem).wait()
            return 0

        lax.fori_loop(0, CH, drain_row, 0, unroll=2)

    def compute_chunk(c, buf):
        def row_body(i, _):
            t, s = i // 8, i % 8
            acc = jnp.zeros((LANES,), jnp.float32)
            for k in range(EMBED // LANES):
                a = arows_v[buf, t, s, pl.ds(k * LANES, LANES)]
                b = brows_v[buf, t, s, pl.ds(k * LANES, LANES)]
                acc = acc + a * b
            plsc.store_compressed(out_v.at[pl.ds(c * CH + i, LANES)],
                                  plsc.cumsum(acc), mask=last_lane)
            return 0

        lax.fori_loop(0, CH, row_body, 0, unroll=4)

    issue_chunk(0, 0)
    for c in range(NCHUNK):
        nxt = (c + 1) % 2
        if c + 1 < NCHUNK:
            issue_chunk(c + 1, nxt)
        drain_chunk(c % 2)
        compute_chunk(c, c % 2)

    pltpu.sync_copy(out_v.at[pl.ds(0, BPW)], out_hbm.at[pl.ds(base, BPW)])


def kernel(center_idx, context_idx, in_embedding, out_embedding):
    return _w2v_sc(center_idx.astype(jnp.int32), context_idx.astype(jnp.int32),
                   in_embedding, out_embedding)
